# baseline (speedup 1.0000x reference)
import functools

import jax
import jax.numpy as jnp
from jax import lax
from jax.experimental import pallas as pl
from jax.experimental.pallas import tpu as pltpu
from jax.experimental.pallas import tpu_sc as plsc

NWORDS = 1000000
NTAGS = 1000
EMB = 64
HID = 512
L = 16384
NW = 32
B_PER_W = L // NW
N_CHUNK = 4
CHUNK = B_PER_W // N_CHUNK


def _sc_body(words_hbm, emb_hbm, out_hbm, idx_v, rows_v, acc_v, sem):
    cid = lax.axis_index("c")
    sid = lax.axis_index("s")
    wid = sid * 2 + cid

    pltpu.sync_copy(words_hbm.at[pl.ds(wid * N_CHUNK, N_CHUNK)], idx_v)

    copies = []
    for k in range(N_CHUNK):
        copies.append(
            pltpu.async_copy(
                emb_hbm.at[idx_v.at[k]],
                rows_v.at[pl.ds(k * CHUNK, CHUNK)],
                sem,
            )
        )
    for c in copies:
        c.wait()

    zero = jnp.zeros((16,), jnp.float32)

    @pl.loop(0, B_PER_W, init_carry=(zero, zero, zero, zero), unroll=8)
    def _acc(j, carry):
        a0, a1, a2, a3 = carry
        a0 = a0 + rows_v[j, pl.ds(0, 16)]
        a1 = a1 + rows_v[j, pl.ds(16, 16)]
        a2 = a2 + rows_v[j, pl.ds(32, 16)]
        a3 = a3 + rows_v[j, pl.ds(48, 16)]
        return a0, a1, a2, a3

    a0, a1, a2, a3 = _acc
    acc_v[pl.ds(0, 16)] = a0
    acc_v[pl.ds(16, 16)] = a1
    acc_v[pl.ds(32, 16)] = a2
    acc_v[pl.ds(48, 16)] = a3

    pltpu.sync_copy(acc_v, out_hbm.at[wid])


def _make_sc_gather_sum():
    mesh = plsc.VectorSubcoreMesh(core_axis_name="c", subcore_axis_name="s")
    return pl.kernel(
        _sc_body,
        out_type=jax.ShapeDtypeStruct((NW, EMB), jnp.float32),
        mesh=mesh,
        scratch_types=[
            pltpu.VMEM((N_CHUNK, CHUNK), jnp.int32),
            pltpu.VMEM((B_PER_W, EMB), jnp.float32),
            pltpu.VMEM((EMB,), jnp.float32),
            pltpu.SemaphoreType.DMA,
        ],
        compiler_params=pltpu.CompilerParams(use_tc_tiling_on_sc=False),
    )


def _mlp_body(parts_ref, w0_ref, b0_ref, w1_ref, b1_ref, wout_ref, bout_ref,
              out_ref):
    s = jnp.sum(parts_ref[...], axis=0, keepdims=True)
    h = jnp.tanh(
        lax.dot_general(s, w0_ref[...], (((1,), (1,)), ((), ())),
                        preferred_element_type=jnp.float32) + b0_ref[...])
    h = jnp.tanh(
        lax.dot_general(h, w1_ref[...], (((1,), (1,)), ((), ())),
                        preferred_element_type=jnp.float32) + b1_ref[...])
    out_ref[...] = lax.dot_general(
        h, wout_ref[...], (((1,), (1,)), ((), ())),
        preferred_element_type=jnp.float32) + bout_ref[...]


_mlp_call = pl.pallas_call(
    _mlp_body,
    out_shape=jax.ShapeDtypeStruct((1, NTAGS), jnp.float32),
)


@jax.jit
def kernel(words, emb, W0, b0, W1, b1, Wout, bout):
    words2 = words.astype(jnp.int32).reshape(NW * N_CHUNK, CHUNK)
    partials = _make_sc_gather_sum()(words2, emb)
    return _mlp_call(partials, W0, b0.reshape(1, HID), W1, b1.reshape(1, HID),
                     Wout, bout.reshape(1, NTAGS))
